# baseline (speedup 1.0000x reference)
import functools

import jax
import jax.numpy as jnp
from jax import lax
from jax.experimental import pallas as pl
from jax.experimental.pallas import tpu as pltpu
from jax.experimental.pallas import tpu_sc as plsc

N = 100000
E = 1600000
NUM_GRAPHS = 16
K_POOL = 64

NPAD = 100352
ROWS = NPAD // 128
EPAD = 16 * NPAD
ERWS = EPAD // 128
CHUNKS = 196
TROWS = NPAD // 16
TSTAGE = 392
BN_BLK = 2048
GRID = NPAD // BN_BLK


def _make_seg_kernel(P):
    PP = P // 2
    mesh = plsc.VectorSubcoreMesh(core_axis_name="c", subcore_axis_name="s")

    @functools.partial(
        pl.kernel, mesh=mesh,
        compiler_params=pltpu.CompilerParams(use_tc_tiling_on_sc=False),
        out_type=jax.ShapeDtypeStruct((P, NPAD, 16), jnp.float32),
        scratch_types=[
            pltpu.VMEM((2, 4, 128), jnp.int32),
            pltpu.VMEM((2, 4, 128), jnp.int32),
            pltpu.VMEM((2, 4, 128, 16), jnp.float32),
            pltpu.VMEM((TSTAGE, 16), jnp.float32),
            pltpu.VMEM_SHARED((NPAD, 16), jnp.float32),
            pltpu.SemaphoreType.DMA,
            pltpu.SemaphoreType.DMA,
        ],
    )
    def seg(xp, src2d, dst2d, zeros_hbm, out,
            src_v, dst_v, rows_v, stage_v, acc, gsem, ssem):
        c = lax.axis_index("c")
        t = lax.axis_index("s")
        row_base = t * (CHUNKS * 4)
        out_base = t * TROWS

        def g_wait(p, b):
            for jj in range(4):
                pltpu.make_async_copy(xp.at[p].at[src_v.at[b, jj]],
                                      rows_v.at[b, jj], gsem).wait()

        def s_wait(b):
            for jj in range(4):
                pltpu.make_async_copy(rows_v.at[b, jj],
                                      acc.at[dst_v.at[b, jj]], ssem).wait()

        def idx_load(k, b):
            r0 = row_base + k * 4
            pltpu.sync_copy(src2d.at[pl.ds(r0, 4)], src_v.at[b])
            pltpu.sync_copy(dst2d.at[pl.ds(r0, 4)], dst_v.at[b])

        def g_fire(p, b):
            for jj in range(4):
                pltpu.async_copy(xp.at[p].at[src_v.at[b, jj]],
                                 rows_v.at[b, jj], gsem)

        def s_fire(b):
            for jj in range(4):
                pltpu.async_copy(rows_v.at[b, jj], acc.at[dst_v.at[b, jj]],
                                 ssem, add=True)

        for j in range(PP):
            p = c * PP + j
            pltpu.sync_copy(zeros_hbm, stage_v)
            for kk in range(TROWS // TSTAGE):
                pltpu.sync_copy(stage_v,
                                acc.at[pl.ds(out_base + kk * TSTAGE, TSTAGE)])
            plsc.subcore_barrier()

            idx_load(0, 0)
            g_fire(p, 0)

            def pair(m, carry):
                for b in (0, 1):
                    k = m * 2 + b
                    nb = 1 - b
                    g_wait(p, b)
                    s_fire(b)

                    @pl.when(k < CHUNKS - 1)
                    def _prefetch():
                        @pl.when(k > 0)
                        def _drain_prev():
                            s_wait(nb)
                        idx_load(k + 1, nb)
                        g_fire(p, nb)
                return carry

            lax.fori_loop(0, CHUNKS // 2, pair, 0)
            s_wait(0)
            s_wait(1)
            plsc.subcore_barrier()
            for kk in range(TROWS // TSTAGE):
                o0 = out_base + kk * TSTAGE
                pltpu.sync_copy(acc.at[pl.ds(o0, TSTAGE)], stage_v)
                pltpu.sync_copy(stage_v, out.at[p, pl.ds(o0, TSTAGE)])

    return seg


_seg2 = _make_seg_kernel(2)
_seg8 = _make_seg_kernel(8)


def _stage1_kernel(agg1_ref, x_ref, wl_ref, bl_ref, wr_ref, wr2_ref,
                   h_ref, q_ref):
    agg = agg1_ref[...]
    cnt = jnp.maximum(agg[:, 27:28], 1.0)
    mean = agg / cnt
    h = jax.nn.relu(
        jnp.dot(mean, wl_ref[...], preferred_element_type=jnp.float32)
        + bl_ref[...]
        + jnp.dot(x_ref[...], wr_ref[...], preferred_element_type=jnp.float32))
    h_ref[...] = h
    q_ref[...] = jnp.dot(h, wr2_ref[...], preferred_element_type=jnp.float32)


def _stage2_kernel(agg2a_ref, cnta_ref, qa_ref, agg2b_ref, cntb_ref, qb_ref,
                   wl2_ref, bl2_ref, d_ref):
    def enc_out(agg2, cnt, q):
        mean = agg2 / jnp.maximum(cnt, 1.0)
        return (jnp.dot(mean, wl2_ref[...],
                        preferred_element_type=jnp.float32)
                + bl2_ref[...] + q)

    h1 = enc_out(agg2a_ref[...], cnta_ref[...], qa_ref[...])
    h2 = enc_out(agg2b_ref[...], cntb_ref[...], qb_ref[...])
    diff = h1 - h2 + 1e-6
    d_ref[...] = jnp.sqrt(jnp.sum(diff * diff, axis=-1, keepdims=True))


def _pool_head_kernel(sgn_ref, skey_ref, batch_ref,
                      w1_ref, b1_ref, g1_ref, be1_ref,
                      w2_ref, b2_ref, g2_ref, be2_ref,
                      w3_ref, b3_ref, out_ref):
    sgn = sgn_ref[0]
    skey = skey_ref[...]
    batch = batch_ref[...]
    neg_inf = jnp.float32(-jnp.inf)
    flat_iota = (jax.lax.broadcasted_iota(jnp.int32, skey.shape, 0) * 128
                 + jax.lax.broadcasted_iota(jnp.int32, skey.shape, 1))
    kiota = jax.lax.broadcasted_iota(jnp.int32, (1, K_POOL), 1)
    giota = jax.lax.broadcasted_iota(jnp.int32, (NUM_GRAPHS, K_POOL), 0)

    def graph_body(g, acc):
        key0 = jnp.where(batch == g, skey, neg_inf)

        def k_body(k, carry):
            key, row = carry
            m = jnp.max(key)
            idx = jnp.min(jnp.where(key == m, flat_iota, jnp.int32(2**31 - 1)))
            key = jnp.where(flat_iota == idx, neg_inf, key)
            row = jnp.where(kiota == k, m, row)
            return key, row

        _, row = jax.lax.fori_loop(
            0, K_POOL, k_body,
            (key0, jnp.zeros((1, K_POOL), jnp.float32)))
        return jnp.where(giota == g, row, acc)

    keys = jax.lax.fori_loop(
        0, NUM_GRAPHS, graph_body,
        jnp.zeros((NUM_GRAPHS, K_POOL), jnp.float32))

    x = (sgn * keys) * jnp.tanh(keys)
    x = jnp.dot(x, w1_ref[...], preferred_element_type=jnp.float32) + b1_ref[...]
    mu = jnp.mean(x, axis=0)
    var = jnp.mean((x - mu) ** 2, axis=0)
    x = (x - mu) / jnp.sqrt(var + 1e-5) * g1_ref[...] + be1_ref[...]
    x = jax.nn.relu(x)
    x = jnp.dot(x, w2_ref[...], preferred_element_type=jnp.float32) + b2_ref[...]
    mu = jnp.mean(x, axis=0)
    var = jnp.mean((x - mu) ** 2, axis=0)
    x = (x - mu) / jnp.sqrt(var + 1e-5) * g2_ref[...] + be2_ref[...]
    x = jax.nn.relu(x)
    x = jnp.dot(x, w3_ref[...], preferred_element_type=jnp.float32) + b3_ref[...]
    out_ref[...] = jax.nn.sigmoid(x)


def _stage1(agg1, xpad, Wl1p, bl1, Wr1p, Wr2):
    blk = lambda i: (i, 0)
    zer = lambda i: (0, 0)
    return pl.pallas_call(
        _stage1_kernel,
        grid=(GRID,),
        in_specs=[
            pl.BlockSpec((BN_BLK, 32), blk),
            pl.BlockSpec((BN_BLK, 32), blk),
            pl.BlockSpec((32, 128), zer),
            pl.BlockSpec((128,), lambda i: (0,)),
            pl.BlockSpec((32, 128), zer),
            pl.BlockSpec((128, 64), zer),
        ],
        out_specs=[
            pl.BlockSpec((BN_BLK, 128), blk),
            pl.BlockSpec((BN_BLK, 64), blk),
        ],
        out_shape=[
            jax.ShapeDtypeStruct((NPAD, 128), jnp.float32),
            jax.ShapeDtypeStruct((NPAD, 64), jnp.float32),
        ],
    )(agg1, xpad, Wl1p, bl1, Wr1p, Wr2)


def _stage2(agg2a, cnta, qa, agg2b, cntb, qb, Wl2, bl2):
    blk = lambda i: (i, 0)
    zer = lambda i: (0, 0)
    return pl.pallas_call(
        _stage2_kernel,
        grid=(GRID,),
        in_specs=[
            pl.BlockSpec((BN_BLK, 128), blk),
            pl.BlockSpec((BN_BLK, 1), blk),
            pl.BlockSpec((BN_BLK, 64), blk),
            pl.BlockSpec((BN_BLK, 128), blk),
            pl.BlockSpec((BN_BLK, 1), blk),
            pl.BlockSpec((BN_BLK, 64), blk),
            pl.BlockSpec((128, 64), zer),
            pl.BlockSpec((64,), lambda i: (0,)),
        ],
        out_specs=pl.BlockSpec((BN_BLK, 1), blk),
        out_shape=jax.ShapeDtypeStruct((NPAD, 1), jnp.float32),
    )(agg2a, cnta, qa, agg2b, cntb, qb, Wl2, bl2)


def _pad_edges(edge_index):
    src = edge_index[0]
    dst = edge_index[1]
    pad = EPAD - E
    it = jnp.arange(pad, dtype=jnp.int32)
    src = jnp.concatenate([src, it % N])
    dst = jnp.concatenate([dst, N + it % (NPAD - N)])
    return src.reshape(ERWS, 128), dst.reshape(ERWS, 128)


def _parts(a, P):
    return a.reshape(NPAD, P, 16).transpose(1, 0, 2)


def kernel(x1, x2, edge_index1, edge_index2, batch1, Wl1, bl1, Wr1,
           Wl2, bl2, Wr2, w_pool, lin1_W, lin1_b, bn1_g, bn1_b,
           lin2_W, lin2_b, bn2_g, bn2_b, lin3_W, lin3_b):
    zeros_hbm = jnp.zeros((TSTAGE, 16), jnp.float32)
    Wl1p = jnp.pad(Wl1, ((0, 5), (0, 0)))
    Wr1p = jnp.pad(Wr1, ((0, 5), (0, 0)))

    def prep_x(x):
        xp = jnp.pad(x, ((0, NPAD - N), (0, 5)))
        return xp.at[:N, 27].set(1.0)

    outs = []
    for x, ei in ((x1, edge_index1), (x2, edge_index2)):
        xpad = prep_x(x)
        src2d, dst2d = _pad_edges(ei)
        agg1p = _seg2(_parts(xpad, 2), src2d, dst2d, zeros_hbm)
        agg1 = agg1p.transpose(1, 0, 2).reshape(NPAD, 32)
        h, q = _stage1(agg1, xpad, Wl1p, bl1, Wr1p, Wr2)
        agg2p = _seg8(_parts(h, 8), src2d, dst2d, zeros_hbm)
        agg2 = agg2p.transpose(1, 0, 2).reshape(NPAD, 128)
        cnt = agg1[:, 27:28]
        outs.append((agg2, cnt, q))

    (a1, c1, q1), (a2, c2, q2) = outs
    d = _stage2(a1, c1, q1, a2, c2, q2, Wl2, bl2)

    sgn = jnp.sign(w_pool[0])[None]
    skey = (sgn[0] * d[:, 0]).reshape(ROWS, 128)
    batch_pad = jnp.pad(batch1, (0, NPAD - N),
                        constant_values=NUM_GRAPHS).reshape(ROWS, 128)
    return pl.pallas_call(
        _pool_head_kernel,
        in_specs=[pl.BlockSpec(memory_space=pltpu.SMEM)]
        + [pl.BlockSpec()] * 12,
        out_shape=jax.ShapeDtypeStruct((NUM_GRAPHS, 1), jnp.float32),
    )(sgn, skey, batch_pad,
      lin1_W, lin1_b, bn1_g, bn1_b,
      lin2_W, lin2_b, bn2_g, bn2_b,
      lin3_W, lin3_b)


# speedup vs baseline: 1.2651x; 1.2651x over previous
import functools

import jax
import jax.numpy as jnp
from jax import lax
from jax.experimental import pallas as pl
from jax.experimental.pallas import tpu as pltpu
from jax.experimental.pallas import tpu_sc as plsc

N = 100000
E = 1600000
NUM_GRAPHS = 16
K_POOL = 64

NPAD = 100352
ROWS = NPAD // 128
EPAD = 16 * NPAD
ERWS = EPAD // 128
CHUNKS = 98
TROWS = NPAD // 16
TSTAGE = 392
BN_BLK = 2048
GRID = NPAD // BN_BLK


def _make_seg_kernel(P):
    PP = P // 2
    mesh = plsc.VectorSubcoreMesh(core_axis_name="c", subcore_axis_name="s")

    @functools.partial(
        pl.kernel, mesh=mesh,
        compiler_params=pltpu.CompilerParams(use_tc_tiling_on_sc=False),
        out_type=jax.ShapeDtypeStruct((P, NPAD, 16), jnp.float32),
        scratch_types=[
            pltpu.VMEM((2, 8, 128), jnp.int32),
            pltpu.VMEM((2, 8, 128), jnp.int32),
            pltpu.VMEM((8, 128, 16), jnp.float32),
            pltpu.VMEM((TSTAGE, 16), jnp.float32),
            pltpu.VMEM_SHARED((NPAD, 16), jnp.float32),
            pltpu.SemaphoreType.DMA,
            pltpu.SemaphoreType.DMA,
        ],
    )
    def seg(xp, src2d, dst2d, zeros_hbm, out,
            src_v, dst_v, rows_v, stage_v, acc, gsem, ssem):
        c = lax.axis_index("c")
        t = lax.axis_index("s")
        row_base = t * (CHUNKS * 4)
        out_base = t * TROWS

        def g_wait(p, b):
            for jj in range(8):
                pltpu.make_async_copy(xp.at[p].at[src_v.at[b, jj]],
                                      rows_v.at[jj], gsem).wait()

        def s_wait(b):
            for jj in range(8):
                pltpu.make_async_copy(rows_v.at[jj],
                                      acc.at[dst_v.at[b, jj]], ssem).wait()

        def idx_load(k, b):
            r0 = row_base + k * 8
            pltpu.sync_copy(src2d.at[pl.ds(r0, 8)], src_v.at[b])
            pltpu.sync_copy(dst2d.at[pl.ds(r0, 8)], dst_v.at[b])

        def g_fire(p, b):
            for jj in range(8):
                pltpu.async_copy(xp.at[p].at[src_v.at[b, jj]],
                                 rows_v.at[jj], gsem)

        def s_fire(b):
            for jj in range(8):
                pltpu.async_copy(rows_v.at[jj], acc.at[dst_v.at[b, jj]],
                                 ssem, add=True)

        for j in range(PP):
            p = c * PP + j
            pltpu.sync_copy(zeros_hbm, stage_v)
            for kk in range(TROWS // TSTAGE):
                pltpu.sync_copy(stage_v,
                                acc.at[pl.ds(out_base + kk * TSTAGE, TSTAGE)])
            plsc.subcore_barrier()

            idx_load(0, 0)
            g_fire(p, 0)

            def pair(m, carry):
                for b in (0, 1):
                    k = m * 2 + b
                    nb = 1 - b
                    g_wait(p, b)
                    s_fire(b)

                    @pl.when(k < CHUNKS - 1)
                    def _prefetch():
                        idx_load(k + 1, nb)
                    s_wait(b)

                    @pl.when(k < CHUNKS - 1)
                    def _next():
                        g_fire(p, nb)
                return carry

            lax.fori_loop(0, CHUNKS // 2, pair, 0)
            plsc.subcore_barrier()
            for kk in range(TROWS // TSTAGE):
                o0 = out_base + kk * TSTAGE
                pltpu.sync_copy(acc.at[pl.ds(o0, TSTAGE)], stage_v)
                pltpu.sync_copy(stage_v, out.at[p, pl.ds(o0, TSTAGE)])

    return seg


_seg2 = _make_seg_kernel(2)
_seg8 = _make_seg_kernel(8)


def _stage1_kernel(agg1_ref, x_ref, wl_ref, bl_ref, wr_ref, wr2_ref,
                   h_ref, q_ref):
    agg = agg1_ref[...]
    cnt = jnp.maximum(agg[:, 27:28], 1.0)
    mean = agg / cnt
    h = jax.nn.relu(
        jnp.dot(mean, wl_ref[...], preferred_element_type=jnp.float32)
        + bl_ref[...]
        + jnp.dot(x_ref[...], wr_ref[...], preferred_element_type=jnp.float32))
    h_ref[...] = h
    q_ref[...] = jnp.dot(h, wr2_ref[...], preferred_element_type=jnp.float32)


def _stage2_kernel(agg2a_ref, cnta_ref, qa_ref, agg2b_ref, cntb_ref, qb_ref,
                   wl2_ref, bl2_ref, d_ref):
    def enc_out(agg2, cnt, q):
        mean = agg2 / jnp.maximum(cnt, 1.0)
        return (jnp.dot(mean, wl2_ref[...],
                        preferred_element_type=jnp.float32)
                + bl2_ref[...] + q)

    h1 = enc_out(agg2a_ref[...], cnta_ref[...], qa_ref[...])
    h2 = enc_out(agg2b_ref[...], cntb_ref[...], qb_ref[...])
    diff = h1 - h2 + 1e-6
    d_ref[...] = jnp.sqrt(jnp.sum(diff * diff, axis=-1, keepdims=True))


def _pool_head_kernel(sgn_ref, skey_ref, batch_ref,
                      w1_ref, b1_ref, g1_ref, be1_ref,
                      w2_ref, b2_ref, g2_ref, be2_ref,
                      w3_ref, b3_ref, out_ref):
    sgn = sgn_ref[0]
    skey = skey_ref[...]
    batch = batch_ref[...]
    neg_inf = jnp.float32(-jnp.inf)
    flat_iota = (jax.lax.broadcasted_iota(jnp.int32, skey.shape, 0) * 128
                 + jax.lax.broadcasted_iota(jnp.int32, skey.shape, 1))
    kiota = jax.lax.broadcasted_iota(jnp.int32, (1, K_POOL), 1)
    giota = jax.lax.broadcasted_iota(jnp.int32, (NUM_GRAPHS, K_POOL), 0)

    def graph_body(g, acc):
        key0 = jnp.where(batch == g, skey, neg_inf)

        def k_body(k, carry):
            key, row = carry
            m = jnp.max(key)
            idx = jnp.min(jnp.where(key == m, flat_iota, jnp.int32(2**31 - 1)))
            key = jnp.where(flat_iota == idx, neg_inf, key)
            row = jnp.where(kiota == k, m, row)
            return key, row

        _, row = jax.lax.fori_loop(
            0, K_POOL, k_body,
            (key0, jnp.zeros((1, K_POOL), jnp.float32)))
        return jnp.where(giota == g, row, acc)

    keys = jax.lax.fori_loop(
        0, NUM_GRAPHS, graph_body,
        jnp.zeros((NUM_GRAPHS, K_POOL), jnp.float32))

    x = (sgn * keys) * jnp.tanh(keys)
    x = jnp.dot(x, w1_ref[...], preferred_element_type=jnp.float32) + b1_ref[...]
    mu = jnp.mean(x, axis=0)
    var = jnp.mean((x - mu) ** 2, axis=0)
    x = (x - mu) / jnp.sqrt(var + 1e-5) * g1_ref[...] + be1_ref[...]
    x = jax.nn.relu(x)
    x = jnp.dot(x, w2_ref[...], preferred_element_type=jnp.float32) + b2_ref[...]
    mu = jnp.mean(x, axis=0)
    var = jnp.mean((x - mu) ** 2, axis=0)
    x = (x - mu) / jnp.sqrt(var + 1e-5) * g2_ref[...] + be2_ref[...]
    x = jax.nn.relu(x)
    x = jnp.dot(x, w3_ref[...], preferred_element_type=jnp.float32) + b3_ref[...]
    out_ref[...] = jax.nn.sigmoid(x)


def _stage1(agg1, xpad, Wl1p, bl1, Wr1p, Wr2):
    blk = lambda i: (i, 0)
    zer = lambda i: (0, 0)
    return pl.pallas_call(
        _stage1_kernel,
        grid=(GRID,),
        in_specs=[
            pl.BlockSpec((BN_BLK, 32), blk),
            pl.BlockSpec((BN_BLK, 32), blk),
            pl.BlockSpec((32, 128), zer),
            pl.BlockSpec((128,), lambda i: (0,)),
            pl.BlockSpec((32, 128), zer),
            pl.BlockSpec((128, 64), zer),
        ],
        out_specs=[
            pl.BlockSpec((BN_BLK, 128), blk),
            pl.BlockSpec((BN_BLK, 64), blk),
        ],
        out_shape=[
            jax.ShapeDtypeStruct((NPAD, 128), jnp.float32),
            jax.ShapeDtypeStruct((NPAD, 64), jnp.float32),
        ],
    )(agg1, xpad, Wl1p, bl1, Wr1p, Wr2)


def _stage2(agg2a, cnta, qa, agg2b, cntb, qb, Wl2, bl2):
    blk = lambda i: (i, 0)
    zer = lambda i: (0, 0)
    return pl.pallas_call(
        _stage2_kernel,
        grid=(GRID,),
        in_specs=[
            pl.BlockSpec((BN_BLK, 128), blk),
            pl.BlockSpec((BN_BLK, 1), blk),
            pl.BlockSpec((BN_BLK, 64), blk),
            pl.BlockSpec((BN_BLK, 128), blk),
            pl.BlockSpec((BN_BLK, 1), blk),
            pl.BlockSpec((BN_BLK, 64), blk),
            pl.BlockSpec((128, 64), zer),
            pl.BlockSpec((64,), lambda i: (0,)),
        ],
        out_specs=pl.BlockSpec((BN_BLK, 1), blk),
        out_shape=jax.ShapeDtypeStruct((NPAD, 1), jnp.float32),
    )(agg2a, cnta, qa, agg2b, cntb, qb, Wl2, bl2)


def _pad_edges(edge_index):
    src = edge_index[0]
    dst = edge_index[1]
    pad = EPAD - E
    it = jnp.arange(pad, dtype=jnp.int32)
    src = jnp.concatenate([src, it % N])
    dst = jnp.concatenate([dst, N + it % (NPAD - N)])
    return src.reshape(ERWS, 128), dst.reshape(ERWS, 128)


def _parts(a, P):
    return a.reshape(NPAD, P, 16).transpose(1, 0, 2)


def kernel(x1, x2, edge_index1, edge_index2, batch1, Wl1, bl1, Wr1,
           Wl2, bl2, Wr2, w_pool, lin1_W, lin1_b, bn1_g, bn1_b,
           lin2_W, lin2_b, bn2_g, bn2_b, lin3_W, lin3_b):
    zeros_hbm = jnp.zeros((TSTAGE, 16), jnp.float32)
    Wl1p = jnp.pad(Wl1, ((0, 5), (0, 0)))
    Wr1p = jnp.pad(Wr1, ((0, 5), (0, 0)))

    def prep_x(x):
        xp = jnp.pad(x, ((0, NPAD - N), (0, 5)))
        return xp.at[:N, 27].set(1.0)

    outs = []
    for x, ei in ((x1, edge_index1), (x2, edge_index2)):
        xpad = prep_x(x)
        src2d, dst2d = _pad_edges(ei)
        agg1p = _seg2(_parts(xpad, 2), src2d, dst2d, zeros_hbm)
        agg1 = agg1p.transpose(1, 0, 2).reshape(NPAD, 32)
        h, q = _stage1(agg1, xpad, Wl1p, bl1, Wr1p, Wr2)
        agg2p = _seg8(_parts(h, 8), src2d, dst2d, zeros_hbm)
        agg2 = agg2p.transpose(1, 0, 2).reshape(NPAD, 128)
        cnt = agg1[:, 27:28]
        outs.append((agg2, cnt, q))

    (a1, c1, q1), (a2, c2, q2) = outs
    d = _stage2(a1, c1, q1, a2, c2, q2, Wl2, bl2)

    sgn = jnp.sign(w_pool[0])[None]
    skey = (sgn[0] * d[:, 0]).reshape(ROWS, 128)
    batch_pad = jnp.pad(batch1, (0, NPAD - N),
                        constant_values=NUM_GRAPHS).reshape(ROWS, 128)
    return pl.pallas_call(
        _pool_head_kernel,
        in_specs=[pl.BlockSpec(memory_space=pltpu.SMEM)]
        + [pl.BlockSpec()] * 12,
        out_shape=jax.ShapeDtypeStruct((NUM_GRAPHS, 1), jnp.float32),
    )(sgn, skey, batch_pad,
      lin1_W, lin1_b, bn1_g, bn1_b,
      lin2_W, lin2_b, bn2_g, bn2_b,
      lin3_W, lin3_b)
